# baseline (speedup 1.0000x reference)
import functools
import jax
import jax.numpy as jnp
from jax import lax
from jax.experimental import pallas as pl
from jax.experimental.pallas import tpu as pltpu
from jax.experimental.pallas import tpu_sc as plsc

NC = 2
NS = 16
L = 16
NW = NC * NS
B = 16384
D = 64
V = 1000000
BPW = B // NW
CHUNK = 128
HALFW = BPW // 2
NPASS = 2
RC = 1024
TGRID = (V + RC - 1) // RC

_mesh = plsc.VectorSubcoreMesh(core_axis_name="c", subcore_axis_name="s")


def _tr_body(eye_ref, in_ref, out_ref):
    t = lax.dot_general(in_ref[...], eye_ref[...], (((0,), (0,)), ((), ())),
                        preferred_element_type=jnp.float32)
    out_ref[...] = jnp.concatenate(
        [t, jnp.zeros((RC, D), jnp.float32)], axis=1)


_tc_transpose = pl.pallas_call(
    _tr_body,
    grid=(TGRID,),
    in_specs=[
        pl.BlockSpec((D, D), lambda i: (0, 0)),
        pl.BlockSpec((D, RC), lambda i: (0, i)),
    ],
    out_specs=pl.BlockSpec((RC, 2 * D), lambda i: (i, 0)),
    out_shape=jax.ShapeDtypeStruct((TGRID * RC, 2 * D), jnp.float32),
)


@functools.partial(
    pl.kernel,
    out_type=jax.ShapeDtypeStruct((B,), jnp.float32),
    mesh=_mesh,
    compiler_params=pltpu.CompilerParams(needs_layout_passes=False),
    scratch_types=[
        pltpu.VMEM((BPW // CHUNK, CHUNK), jnp.int32),
        pltpu.VMEM((BPW // CHUNK, CHUNK), jnp.int32),
        pltpu.VMEM((BPW // CHUNK, CHUNK), jnp.int32),
        pltpu.VMEM((HALFW, 2 * D), jnp.float32),
        pltpu.VMEM((HALFW, 2 * D), jnp.float32),
        pltpu.VMEM((BPW,), jnp.float32),
        pltpu.SemaphoreType.DMA,
    ],
)
def _mf_rating(user_hbm, item_hbm, u128_hbm, ipair_hbm, out_hbm,
               uidx, iidx, igid, urows, irows, out_v, gsem):
    wid = lax.axis_index("s") * NC + lax.axis_index("c")
    base = wid * BPW
    nchunk = BPW // CHUNK

    for c in range(nchunk):
        pltpu.sync_copy(user_hbm.at[pl.ds(base + c * CHUNK, CHUNK)],
                        uidx.at[c])
        pltpu.sync_copy(item_hbm.at[pl.ds(base + c * CHUNK, CHUNK)],
                        iidx.at[c])

    for c in range(nchunk):
        for k in range(CHUNK // L):
            sl = pl.ds(k * L, L)
            igid[c, sl] = lax.shift_right_logical(iidx[c, sl], 1)

    row_iota = lax.iota(jnp.int32, L)

    def do_pass(p):
        copies = []
        for cc in range(HALFW // CHUNK):
            c = p * (HALFW // CHUNK) + cc
            copies.append(pltpu.async_copy(
                u128_hbm.at[uidx.at[c]],
                urows.at[pl.ds(cc * CHUNK, CHUNK)], gsem))
            copies.append(pltpu.async_copy(
                ipair_hbm.at[igid.at[c]],
                irows.at[pl.ds(cc * CHUNK, CHUNK)], gsem))
        for cp in copies:
            cp.wait()

        def group(g, carry):
            c = p * (HALFW // CHUNK) + g // 8
            sl = pl.ds((g % 8) * L, L)
            ioff = lax.shift_left(jnp.bitwise_and(iidx[c, sl], 1), 6)
            idx_row = g * L + row_iota
            acc = jnp.zeros((L,), jnp.float32)
            for d in range(D):
                dvec = jnp.full((L,), d, jnp.int32)
                u = plsc.load_gather(urows, [idx_row, dvec])
                i = plsc.load_gather(irows, [idx_row, ioff + d])
                acc = acc + u * i
            out_v[pl.ds(p * HALFW + g * L, L)] = acc
            return carry

        lax.fori_loop(0, HALFW // L, group, 0)

    for p in range(NPASS):
        do_pass(p)

    pltpu.sync_copy(out_v, out_hbm.at[pl.ds(base, BPW)])


def kernel(user, item, user_emb, item_emb):
    u128 = _tc_transpose(jnp.eye(D, dtype=jnp.float32), user_emb.T)
    ipair = item_emb.reshape(V // 2, 2 * D)
    return _mf_rating(user, item, u128, ipair)


# speedup vs baseline: 1.1181x; 1.1181x over previous
import functools
import jax
import jax.numpy as jnp
from jax import lax
from jax.experimental import pallas as pl
from jax.experimental.pallas import tpu as pltpu
from jax.experimental.pallas import tpu_sc as plsc

NC = 2
NS = 16
L = 16
NW = NC * NS
B = 16384
D = 64
V = 1000000
BPW = B // NW
CHUNK = 128
HALFW = BPW // 2
NPASS = 2

_mesh = plsc.VectorSubcoreMesh(core_axis_name="c", subcore_axis_name="s")


@functools.partial(
    pl.kernel,
    out_type=jax.ShapeDtypeStruct((B,), jnp.float32),
    mesh=_mesh,
    compiler_params=pltpu.CompilerParams(needs_layout_passes=False),
    scratch_types=[
        pltpu.VMEM((BPW // CHUNK, CHUNK), jnp.int32),
        pltpu.VMEM((BPW // CHUNK, CHUNK), jnp.int32),
        pltpu.VMEM((BPW // CHUNK, CHUNK), jnp.int32),
        pltpu.VMEM((BPW // CHUNK, CHUNK), jnp.int32),
        pltpu.VMEM((HALFW, 2 * D), jnp.float32),
        pltpu.VMEM((HALFW, 2 * D), jnp.float32),
        pltpu.VMEM((BPW,), jnp.float32),
        pltpu.SemaphoreType.DMA,
    ],
)
def _mf_rating(user_hbm, item_hbm, upair_hbm, ipair_hbm, out_hbm,
               uidx, iidx, ugid, igid, urows, irows, out_v, gsem):
    wid = lax.axis_index("s") * NC + lax.axis_index("c")
    base = wid * BPW
    nchunk = BPW // CHUNK

    for c in range(nchunk):
        pltpu.sync_copy(user_hbm.at[pl.ds(base + c * CHUNK, CHUNK)],
                        uidx.at[c])
        pltpu.sync_copy(item_hbm.at[pl.ds(base + c * CHUNK, CHUNK)],
                        iidx.at[c])

    for c in range(nchunk):
        for k in range(CHUNK // L):
            sl = pl.ds(k * L, L)
            ugid[c, sl] = lax.shift_right_logical(uidx[c, sl], 1)
            igid[c, sl] = lax.shift_right_logical(iidx[c, sl], 1)

    row_iota = lax.iota(jnp.int32, L)

    def do_pass(p):
        copies = []
        for cc in range(HALFW // CHUNK):
            c = p * (HALFW // CHUNK) + cc
            copies.append(pltpu.async_copy(
                upair_hbm.at[ugid.at[c]],
                urows.at[pl.ds(cc * CHUNK, CHUNK)], gsem))
            copies.append(pltpu.async_copy(
                ipair_hbm.at[igid.at[c]],
                irows.at[pl.ds(cc * CHUNK, CHUNK)], gsem))
        for cp in copies:
            cp.wait()

        def group(g, carry):
            c = p * (HALFW // CHUNK) + g // 8
            sl = pl.ds((g % 8) * L, L)
            uoff = lax.shift_left(jnp.bitwise_and(uidx[c, sl], 1), 6)
            ioff = lax.shift_left(jnp.bitwise_and(iidx[c, sl], 1), 6)
            idx_row = g * L + row_iota
            acc = jnp.zeros((L,), jnp.float32)
            for d in range(D):
                u = plsc.load_gather(urows, [idx_row, uoff + d])
                i = plsc.load_gather(irows, [idx_row, ioff + d])
                acc = acc + u * i
            out_v[pl.ds(p * HALFW + g * L, L)] = acc
            return carry

        lax.fori_loop(0, HALFW // L, group, 0)

    for p in range(NPASS):
        do_pass(p)

    pltpu.sync_copy(out_v, out_hbm.at[pl.ds(base, BPW)])


def kernel(user, item, user_emb, item_emb):
    upair = user_emb.reshape(V // 2, 2 * D)
    ipair = item_emb.reshape(V // 2, 2 * D)
    return _mf_rating(user, item, upair, ipair)


# speedup vs baseline: 1.3060x; 1.1680x over previous
import functools
import jax
import jax.numpy as jnp
from jax import lax
from jax.experimental import pallas as pl
from jax.experimental.pallas import tpu as pltpu
from jax.experimental.pallas import tpu_sc as plsc

NC = 2
NS = 16
L = 16
NW = NC * NS
B = 16384
D = 64
V = 1000000
BPW = B // NW
CHUNK = 128
HALFW = BPW // 2
NPASS = 2
RC = 1024
TGRID = (V + RC - 1) // RC
VPAD = TGRID * RC

_mesh = plsc.VectorSubcoreMesh(core_axis_name="c", subcore_axis_name="s")


def _tr_body(eye_ref, u_ref, i_ref, out_ref):
    tu = lax.dot_general(u_ref[...], eye_ref[...], (((0,), (0,)), ((), ())),
                         preferred_element_type=jnp.float32)
    ti = lax.dot_general(i_ref[...], eye_ref[...], (((0,), (0,)), ((), ())),
                         preferred_element_type=jnp.float32)
    out_ref[...] = jnp.concatenate([tu, ti], axis=1)


_tc_combine = pl.pallas_call(
    _tr_body,
    grid=(TGRID,),
    in_specs=[
        pl.BlockSpec((D, D), lambda i: (0, 0)),
        pl.BlockSpec((D, RC), lambda i: (0, i)),
        pl.BlockSpec((D, RC), lambda i: (0, i)),
    ],
    out_specs=pl.BlockSpec((RC, 2 * D), lambda i: (i, 0)),
    out_shape=jax.ShapeDtypeStruct((VPAD, 2 * D), jnp.float32),
)


@functools.partial(
    pl.kernel,
    out_type=jax.ShapeDtypeStruct((B,), jnp.float32),
    mesh=_mesh,
    compiler_params=pltpu.CompilerParams(needs_layout_passes=False),
    scratch_types=[
        pltpu.VMEM((BPW // CHUNK, CHUNK), jnp.int32),
        pltpu.VMEM((BPW // CHUNK, CHUNK), jnp.int32),
        pltpu.VMEM((HALFW, 2 * D), jnp.float32),
        pltpu.VMEM((HALFW, 2 * D), jnp.float32),
        pltpu.VMEM((BPW,), jnp.float32),
        pltpu.SemaphoreType.DMA,
    ],
)
def _mf_rating(user_hbm, item_hbm, comb_hbm, out_hbm,
               uidx, iidx, urows, irows, out_v, gsem):
    wid = lax.axis_index("s") * NC + lax.axis_index("c")
    base = wid * BPW
    nchunk = BPW // CHUNK

    for c in range(nchunk):
        pltpu.sync_copy(user_hbm.at[pl.ds(base + c * CHUNK, CHUNK)],
                        uidx.at[c])
        pltpu.sync_copy(item_hbm.at[pl.ds(base + c * CHUNK, CHUNK)],
                        iidx.at[c])

    row_iota = lax.iota(jnp.int32, L)

    def do_pass(p):
        copies = []
        for cc in range(HALFW // CHUNK):
            c = p * (HALFW // CHUNK) + cc
            copies.append(pltpu.async_copy(
                comb_hbm.at[uidx.at[c]],
                urows.at[pl.ds(cc * CHUNK, CHUNK)], gsem))
            copies.append(pltpu.async_copy(
                comb_hbm.at[iidx.at[c]],
                irows.at[pl.ds(cc * CHUNK, CHUNK)], gsem))
        for cp in copies:
            cp.wait()

        def group(g, carry):
            idx_row = g * L + row_iota
            acc = jnp.zeros((L,), jnp.float32)
            for d in range(D):
                dvec = jnp.full((L,), d, jnp.int32)
                u = plsc.load_gather(urows, [idx_row, dvec])
                i = plsc.load_gather(irows, [idx_row, dvec + D])
                acc = acc + u * i
            out_v[pl.ds(p * HALFW + g * L, L)] = acc
            return carry

        lax.fori_loop(0, HALFW // L, group, 0)

    for p in range(NPASS):
        do_pass(p)

    pltpu.sync_copy(out_v, out_hbm.at[pl.ds(base, BPW)])


def kernel(user, item, user_emb, item_emb):
    comb = _tc_combine(jnp.eye(D, dtype=jnp.float32), user_emb.T, item_emb.T)
    return _mf_rating(user, item, comb)


# speedup vs baseline: 2.2530x; 1.7251x over previous
import functools
import jax
import jax.numpy as jnp
from jax import lax
from jax.experimental import pallas as pl
from jax.experimental.pallas import tpu as pltpu
from jax.experimental.pallas import tpu_sc as plsc

NC = 2
NS = 16
L = 16
NW = NC * NS
B = 16384
D = 64
V = 1000000
BPW = B // NW
CHUNK = 128
HALFW = BPW // 2
NPASS = 2
RC = 4096
TGRID = (V + RC - 1) // RC
VPAD = TGRID * RC

_mesh = plsc.VectorSubcoreMesh(core_axis_name="c", subcore_axis_name="s")


def _tr_body(u_ref, i_ref, out_ref):
    tu = jnp.swapaxes(u_ref[...], 0, 1)
    ti = jnp.swapaxes(i_ref[...], 0, 1)
    out_ref[...] = jnp.concatenate([tu, ti], axis=1)


_tc_combine = pl.pallas_call(
    _tr_body,
    grid=(TGRID,),
    in_specs=[
        pl.BlockSpec((D, RC), lambda i: (0, i)),
        pl.BlockSpec((D, RC), lambda i: (0, i)),
    ],
    out_specs=pl.BlockSpec((RC, 2 * D), lambda i: (i, 0)),
    out_shape=jax.ShapeDtypeStruct((VPAD, 2 * D), jnp.float32),
)


@functools.partial(
    pl.kernel,
    out_type=jax.ShapeDtypeStruct((B,), jnp.float32),
    mesh=_mesh,
    compiler_params=pltpu.CompilerParams(needs_layout_passes=False),
    scratch_types=[
        pltpu.VMEM((BPW // CHUNK, CHUNK), jnp.int32),
        pltpu.VMEM((BPW // CHUNK, CHUNK), jnp.int32),
        pltpu.VMEM((HALFW, 2 * D), jnp.float32),
        pltpu.VMEM((HALFW, 2 * D), jnp.float32),
        pltpu.VMEM((BPW,), jnp.float32),
        pltpu.SemaphoreType.DMA,
    ],
)
def _mf_rating(user_hbm, item_hbm, comb_hbm, out_hbm,
               uidx, iidx, urows, irows, out_v, gsem):
    wid = lax.axis_index("s") * NC + lax.axis_index("c")
    base = wid * BPW
    nchunk = BPW // CHUNK

    for c in range(nchunk):
        pltpu.sync_copy(user_hbm.at[pl.ds(base + c * CHUNK, CHUNK)],
                        uidx.at[c])
        pltpu.sync_copy(item_hbm.at[pl.ds(base + c * CHUNK, CHUNK)],
                        iidx.at[c])

    row_iota = lax.iota(jnp.int32, L)

    def do_pass(p):
        copies = []
        for cc in range(HALFW // CHUNK):
            c = p * (HALFW // CHUNK) + cc
            copies.append(pltpu.async_copy(
                comb_hbm.at[uidx.at[c]],
                urows.at[pl.ds(cc * CHUNK, CHUNK)], gsem))
            copies.append(pltpu.async_copy(
                comb_hbm.at[iidx.at[c]],
                irows.at[pl.ds(cc * CHUNK, CHUNK)], gsem))
        for cp in copies:
            cp.wait()

        def group(g, carry):
            idx_row = g * L + row_iota
            acc = jnp.zeros((L,), jnp.float32)
            for d in range(D):
                dvec = jnp.full((L,), d, jnp.int32)
                u = plsc.load_gather(urows, [idx_row, dvec])
                i = plsc.load_gather(irows, [idx_row, dvec + D])
                acc = acc + u * i
            out_v[pl.ds(p * HALFW + g * L, L)] = acc
            return carry

        lax.fori_loop(0, HALFW // L, group, 0)

    for p in range(NPASS):
        do_pass(p)

    pltpu.sync_copy(out_v, out_hbm.at[pl.ds(base, BPW)])


def kernel(user, item, user_emb, item_emb):
    comb = _tc_combine(user_emb.T, item_emb.T)
    return _mf_rating(user, item, comb)
